# baseline (speedup 1.0000x reference)
import functools
import math

import jax
import jax.numpy as jnp
from jax import lax
from jax.experimental import pallas as pl
from jax.experimental.pallas import tpu as pltpu
from jax.experimental.pallas import tpu_sc as plsc

D_MODEL = 1024
SCALE = math.sqrt(D_MODEL)
LANES = 16
VECS_PER_ROW = D_MODEL // LANES


@functools.lru_cache(maxsize=None)
def _build_sc_embed(B, num_cores, num_subcores, C, NBUF):
    NW = num_cores * num_subcores
    b_per_w = B // NW
    n_chunks = b_per_w // C
    mesh = plsc.VectorSubcoreMesh(core_axis_name="c", subcore_axis_name="s")

    @functools.partial(
        pl.kernel,
        mesh=mesh,
        out_type=jax.ShapeDtypeStruct((B, D_MODEL), jnp.float32),
        scratch_types=[
            pltpu.VMEM((b_per_w,), jnp.int32),
            *[pltpu.VMEM((C, D_MODEL), jnp.float32) for _ in range(NBUF)],
            *[pltpu.SemaphoreType.DMA for _ in range(2 * NBUF)],
        ],
    )
    def sc_embed(idx_hbm, table_hbm, out_hbm, idx_v, *bufs_and_sems):
        rows = bufs_and_sems[:NBUF]
        gsem = bufs_and_sems[NBUF : 2 * NBUF]
        ssem = bufs_and_sems[2 * NBUF : 3 * NBUF]

        wid = lax.axis_index("s") * num_cores + lax.axis_index("c")
        base = wid * b_per_w
        pltpu.sync_copy(idx_hbm.at[pl.ds(base, b_per_w)], idx_v)

        def start_gather(g):
            b = g % NBUF
            return pltpu.async_copy(
                table_hbm.at[idx_v.at[pl.ds(g * C, C)]], rows[b], gsem[b]
            )

        def start_scatter(g):
            b = g % NBUF
            return pltpu.async_copy(
                rows[b], out_hbm.at[pl.ds(base + g * C, C)], ssem[b]
            )

        gathers = [None] * n_chunks
        scatters = [None] * n_chunks
        drained = [False] * n_chunks
        for g in range(min(NBUF - 1, n_chunks)):
            gathers[g] = start_gather(g)

        for g in range(n_chunks):
            b = g % NBUF
            gathers[g].wait()

            nxt = g + NBUF - 1
            if nxt < n_chunks:
                pg = nxt - NBUF
                if pg >= 0:
                    scatters[pg].wait()
                    drained[pg] = True
                gathers[nxt] = start_gather(nxt)

            def row_body(r, carry, rv=rows[b]):
                for c in range(VECS_PER_ROW):
                    s = pl.ds(c * LANES, LANES)
                    rv[r, s] = rv[r, s] * SCALE
                return carry

            lax.fori_loop(0, C, row_body, 0)

            scatters[g] = start_scatter(g)

        for g in range(n_chunks):
            if scatters[g] is not None and not drained[g]:
                scatters[g].wait()

    return sc_embed


def kernel(x, table):
    B = x.shape[0] * x.shape[1]
    idx = x.reshape(B).astype(jnp.int32)
    out = _build_sc_embed(B, 2, 16, 32, 3)(idx, table)
    return out.reshape(x.shape[0], x.shape[1], D_MODEL)


# speedup vs baseline: 1.0765x; 1.0765x over previous
import functools
import math

import jax
import jax.numpy as jnp
from jax import lax
from jax.experimental import pallas as pl
from jax.experimental.pallas import tpu as pltpu
from jax.experimental.pallas import tpu_sc as plsc

D_MODEL = 1024
SCALE = math.sqrt(D_MODEL)
LANES = 16
VECS_PER_ROW = D_MODEL // LANES


@functools.lru_cache(maxsize=None)
def _build_sc_embed(B, num_cores, num_subcores, C, NBUF, GIF):
    NW = num_cores * num_subcores
    b_per_w = B // NW
    n_chunks = b_per_w // C
    assert GIF < NBUF
    mesh = plsc.VectorSubcoreMesh(core_axis_name="c", subcore_axis_name="s")

    @functools.partial(
        pl.kernel,
        mesh=mesh,
        out_type=jax.ShapeDtypeStruct((B, D_MODEL), jnp.float32),
        scratch_types=[
            pltpu.VMEM((b_per_w,), jnp.int32),
            *[pltpu.VMEM((C, D_MODEL), jnp.float32) for _ in range(NBUF)],
            *[pltpu.SemaphoreType.DMA for _ in range(2 * NBUF)],
        ],
    )
    def sc_embed(idx_hbm, table_hbm, out_hbm, idx_v, *bufs_and_sems):
        rows = bufs_and_sems[:NBUF]
        gsem = bufs_and_sems[NBUF : 2 * NBUF]
        ssem = bufs_and_sems[2 * NBUF : 3 * NBUF]

        wid = lax.axis_index("s") * num_cores + lax.axis_index("c")
        base = wid * b_per_w
        pltpu.sync_copy(idx_hbm.at[pl.ds(base, b_per_w)], idx_v)

        def start_gather(g):
            b = g % NBUF
            return pltpu.async_copy(
                table_hbm.at[idx_v.at[pl.ds(g * C, C)]], rows[b], gsem[b]
            )

        def start_scatter(g):
            b = g % NBUF
            return pltpu.async_copy(
                rows[b], out_hbm.at[pl.ds(base + g * C, C)], ssem[b]
            )

        gathers = [None] * n_chunks
        scatters = [None] * n_chunks
        drained = [False] * n_chunks
        for g in range(min(GIF, n_chunks)):
            gathers[g] = start_gather(g)

        for g in range(n_chunks):
            b = g % NBUF
            gathers[g].wait()

            def row_body(r, carry, rv=rows[b]):
                for c in range(VECS_PER_ROW):
                    s = pl.ds(c * LANES, LANES)
                    rv[r, s] = rv[r, s] * SCALE
                return carry

            lax.fori_loop(0, C, row_body, 0)

            scatters[g] = start_scatter(g)

            nxt = g + GIF
            if nxt < n_chunks:
                pg = nxt - NBUF
                if pg >= 0:
                    scatters[pg].wait()
                    drained[pg] = True
                gathers[nxt] = start_gather(nxt)

        for g in range(n_chunks):
            if scatters[g] is not None and not drained[g]:
                scatters[g].wait()

    return sc_embed


def kernel(x, table):
    B = x.shape[0] * x.shape[1]
    idx = x.reshape(B).astype(jnp.int32)
    out = _build_sc_embed(B, 2, 16, 16, 6, 4)(idx, table)
    return out.reshape(x.shape[0], x.shape[1], D_MODEL)


# speedup vs baseline: 1.1133x; 1.0342x over previous
import functools
import math

import jax
import jax.numpy as jnp
from jax import lax
from jax.experimental import pallas as pl
from jax.experimental.pallas import tpu as pltpu
from jax.experimental.pallas import tpu_sc as plsc

D_MODEL = 1024
SCALE = math.sqrt(D_MODEL)
LANES = 16
VECS_PER_ROW = D_MODEL // LANES


@functools.lru_cache(maxsize=None)
def _build_sc_embed(B, num_cores, num_subcores, C, NBUF, GIF):
    NW = num_cores * num_subcores
    b_per_w = B // NW
    assert GIF < NBUF
    ramp = [C // 4, C // 4, C // 2]
    mid = b_per_w - 2 * sum(ramp)
    assert mid > 0 and mid % C == 0
    chunk_len = ramp + [C] * (mid // C) + ramp[::-1]
    chunk_off = [0]
    for L in chunk_len[:-1]:
        chunk_off.append(chunk_off[-1] + L)
    n_chunks = len(chunk_len)
    mesh = plsc.VectorSubcoreMesh(core_axis_name="c", subcore_axis_name="s")

    @functools.partial(
        pl.kernel,
        mesh=mesh,
        out_type=jax.ShapeDtypeStruct((B, D_MODEL), jnp.float32),
        scratch_types=[
            pltpu.VMEM((b_per_w,), jnp.int32),
            *[pltpu.VMEM((C, D_MODEL), jnp.float32) for _ in range(NBUF)],
            *[pltpu.SemaphoreType.DMA for _ in range(2 * NBUF)],
        ],
    )
    def sc_embed(idx_hbm, table_hbm, out_hbm, idx_v, *bufs_and_sems):
        rows = bufs_and_sems[:NBUF]
        gsem = bufs_and_sems[NBUF : 2 * NBUF]
        ssem = bufs_and_sems[2 * NBUF : 3 * NBUF]

        wid = lax.axis_index("s") * num_cores + lax.axis_index("c")
        base = wid * b_per_w
        pltpu.sync_copy(idx_hbm.at[pl.ds(base, b_per_w)], idx_v)

        def start_gather(g):
            b = g % NBUF
            L = chunk_len[g]
            return pltpu.async_copy(
                table_hbm.at[idx_v.at[pl.ds(chunk_off[g], L)]],
                rows[b].at[pl.ds(0, L)],
                gsem[b],
            )

        def start_scatter(g):
            b = g % NBUF
            L = chunk_len[g]
            return pltpu.async_copy(
                rows[b].at[pl.ds(0, L)],
                out_hbm.at[pl.ds(base + chunk_off[g], L)],
                ssem[b],
            )

        gathers = [None] * n_chunks
        scatters = [None] * n_chunks
        drained = [False] * n_chunks
        for g in range(min(GIF, n_chunks)):
            gathers[g] = start_gather(g)

        for g in range(n_chunks):
            b = g % NBUF
            gathers[g].wait()

            def row_body(r, carry, rv=rows[b]):
                for c in range(VECS_PER_ROW):
                    s = pl.ds(c * LANES, LANES)
                    rv[r, s] = rv[r, s] * SCALE
                return carry

            lax.fori_loop(0, chunk_len[g], row_body, 0)

            scatters[g] = start_scatter(g)

            nxt = g + GIF
            if nxt < n_chunks:
                pg = nxt - NBUF
                if pg >= 0:
                    scatters[pg].wait()
                    drained[pg] = True
                gathers[nxt] = start_gather(nxt)

        for g in range(n_chunks):
            if scatters[g] is not None and not drained[g]:
                scatters[g].wait()

    return sc_embed


def kernel(x, table):
    B = x.shape[0] * x.shape[1]
    idx = x.reshape(B).astype(jnp.int32)
    out = _build_sc_embed(B, 2, 16, 32, 3, 2)(idx, table)
    return out.reshape(x.shape[0], x.shape[1], D_MODEL)


# speedup vs baseline: 1.1367x; 1.0210x over previous
import functools
import math

import jax
import jax.numpy as jnp
from jax import lax
from jax.experimental import pallas as pl
from jax.experimental.pallas import tpu as pltpu
from jax.experimental.pallas import tpu_sc as plsc

D_MODEL = 1024
SCALE = math.sqrt(D_MODEL)
LANES = 16
VECS_PER_ROW = D_MODEL // LANES
C = 16
NBUF = 4


@functools.lru_cache(maxsize=None)
def _build_sc_embed(B, num_cores, num_subcores):
    NW = num_cores * num_subcores
    b_per_w = B // NW
    n_chunks = b_per_w // C
    n_groups = n_chunks // NBUF
    assert n_groups * NBUF == n_chunks and n_groups >= 2
    mesh = plsc.VectorSubcoreMesh(core_axis_name="c", subcore_axis_name="s")

    @functools.partial(
        pl.kernel,
        mesh=mesh,
        out_type=jax.ShapeDtypeStruct((B, D_MODEL), jnp.float32),
        scratch_types=[
            pltpu.VMEM((b_per_w,), jnp.int32),
            *[pltpu.VMEM((C, D_MODEL), jnp.float32) for _ in range(NBUF)],
            *[pltpu.SemaphoreType.DMA for _ in range(2 * NBUF)],
        ],
    )
    def sc_embed(idx_hbm, table_hbm, out_hbm, idx_v, *bufs_and_sems):
        rows = bufs_and_sems[:NBUF]
        gsem = bufs_and_sems[NBUF : 2 * NBUF]
        ssem = bufs_and_sems[2 * NBUF : 3 * NBUF]

        wid = lax.axis_index("s") * num_cores + lax.axis_index("c")
        base = wid * b_per_w
        pltpu.sync_copy(idx_hbm.at[pl.ds(base, b_per_w)], idx_v)

        def issue_gather(g, k):
            off = pl.multiple_of(g * C, C)
            return pltpu.async_copy(
                table_hbm.at[idx_v.at[pl.ds(off, C)]], rows[k], gsem[k]
            )

        def issue_scatter(g, k):
            off = pl.multiple_of(base + g * C, C)
            return pltpu.async_copy(
                rows[k], out_hbm.at[pl.ds(off, C)], ssem[k]
            )

        def wait_gather(k):
            pltpu.make_async_copy(
                out_hbm.at[pl.ds(0, C)], rows[k], gsem[k]
            ).wait()

        def wait_scatter(k):
            pltpu.make_async_copy(
                rows[k], out_hbm.at[pl.ds(0, C)], ssem[k]
            ).wait()

        issue_gather(0, 0)
        issue_gather(1, 1)

        def group(j, carry):
            for k in range(NBUF):
                g = j * NBUF + k
                wait_gather(k)

                def row_body(r, cc, rv=rows[k]):
                    for c in range(VECS_PER_ROW):
                        s = pl.ds(c * LANES, LANES)
                        rv[r, s] = rv[r, s] * SCALE
                    return cc

                lax.fori_loop(0, C, row_body, 0)
                issue_scatter(g, k)

                kn = (k + 2) % NBUF
                if k < 2:
                    @pl.when(j >= 1)
                    def _():
                        wait_scatter(kn)

                    issue_gather(g + 2, kn)
                else:
                    wait_scatter(kn)

                    @pl.when(j < n_groups - 1)
                    def _():
                        issue_gather(g + 2, kn)
            return carry

        lax.fori_loop(0, n_groups, group, 0)
        wait_scatter(2)
        wait_scatter(3)

    return sc_embed


def kernel(x, table):
    B = x.shape[0] * x.shape[1]
    idx = x.reshape(B).astype(jnp.int32)
    out = _build_sc_embed(B, 2, 16)(idx, table)
    return out.reshape(x.shape[0], x.shape[1], D_MODEL)


# speedup vs baseline: 1.1616x; 1.0219x over previous
import functools
import math

import jax
import jax.numpy as jnp
from jax import lax
from jax.experimental import pallas as pl
from jax.experimental.pallas import tpu as pltpu
from jax.experimental.pallas import tpu_sc as plsc

D_MODEL = 1024
SCALE = math.sqrt(D_MODEL)
LANES = 16
VECS_PER_ROW = D_MODEL // LANES
C = 32
NBUF = 3


@functools.lru_cache(maxsize=None)
def _build_sc_embed(B, num_cores, num_subcores):
    NW = num_cores * num_subcores
    b_per_w = B // NW
    n_chunks = b_per_w // C
    n_groups = (n_chunks - 1) // NBUF
    assert n_groups * NBUF + 1 == n_chunks and n_groups >= 2
    mesh = plsc.VectorSubcoreMesh(core_axis_name="c", subcore_axis_name="s")

    @functools.partial(
        pl.kernel,
        mesh=mesh,
        out_type=jax.ShapeDtypeStruct((B, D_MODEL), jnp.float32),
        scratch_types=[
            pltpu.VMEM((b_per_w,), jnp.int32),
            *[pltpu.VMEM((C, D_MODEL), jnp.float32) for _ in range(NBUF)],
            *[pltpu.SemaphoreType.DMA for _ in range(2 * NBUF)],
        ],
    )
    def sc_embed(idx_hbm, table_hbm, out_hbm, idx_v, *bufs_and_sems):
        rows = bufs_and_sems[:NBUF]
        gsem = bufs_and_sems[NBUF : 2 * NBUF]
        ssem = bufs_and_sems[2 * NBUF : 3 * NBUF]

        wid = lax.axis_index("s") * num_cores + lax.axis_index("c")
        base = wid * b_per_w
        pltpu.sync_copy(idx_hbm.at[pl.ds(base, b_per_w)], idx_v)

        def issue_gather(g, k):
            off = pl.multiple_of(g * C, C)
            return pltpu.async_copy(
                table_hbm.at[idx_v.at[pl.ds(off, C)]], rows[k], gsem[k]
            )

        def issue_scatter(g, k):
            off = pl.multiple_of(base + g * C, C)
            return pltpu.async_copy(
                rows[k], out_hbm.at[pl.ds(off, C)], ssem[k]
            )

        def wait_gather(k):
            pltpu.make_async_copy(
                out_hbm.at[pl.ds(0, C)], rows[k], gsem[k]
            ).wait()

        def wait_scatter(k):
            pltpu.make_async_copy(
                rows[k], out_hbm.at[pl.ds(0, C)], ssem[k]
            ).wait()

        def scale_chunk(k):
            def row_body(r, cc, rv=rows[k]):
                for c in range(VECS_PER_ROW):
                    s = pl.ds(c * LANES, LANES)
                    rv[r, s] = rv[r, s] * SCALE
                return cc

            lax.fori_loop(0, C, row_body, 0)

        issue_gather(0, 0)
        issue_gather(1, 1)

        wait_gather(0)
        scale_chunk(0)
        issue_scatter(0, 0)
        issue_gather(2, 2)

        def group(j, carry):
            for dg in range(1, NBUF + 1):
                g = j * NBUF + dg
                k = dg % NBUF
                kn = (k + 2) % NBUF
                wait_gather(k)
                scale_chunk(k)
                issue_scatter(g, k)
                wait_scatter(kn)
                if dg == 1:
                    issue_gather(g + 2, kn)
                else:
                    @pl.when(j < n_groups - 1)
                    def _():
                        issue_gather(g + 2, kn)
            return carry

        lax.fori_loop(0, n_groups, group, 0)
        wait_scatter((n_chunks - 1) % NBUF)

    return sc_embed


def kernel(x, table):
    B = x.shape[0] * x.shape[1]
    idx = x.reshape(B).astype(jnp.int32)
    out = _build_sc_embed(B, 2, 16)(idx, table)
    return out.reshape(x.shape[0], x.shape[1], D_MODEL)
